# baseline (speedup 1.0000x reference)
import jax
import jax.numpy as jnp
from jax import lax
from jax.experimental import pallas as pl
from jax.experimental.pallas import tpu as pltpu
from jax.experimental.pallas import tpu_sc as plsc

_NBINS = 15
_C = 96
_H = 512
_W = 512
_R = 64
_PIX = _H * _W
_NW = 32
_PW = _PIX // _NW


def _tc_body(bv_ref, p0_ref, p1_ref, lab_ref, conf_ref, acc_ref, bin_ref):
    x0 = p0_ref[0]
    x1 = p1_ref[0]
    m0 = jnp.max(x0, axis=0)
    m1 = jnp.max(x1, axis=0)
    a0 = jnp.argmax(x0, axis=0).astype(jnp.int32)
    a1 = jnp.argmax(x1, axis=0).astype(jnp.int32)
    conf = jnp.maximum(m0, m1)
    pred = jnp.where(m0 >= m1, a0, a1 + _C // 2)
    acc = (pred == lab_ref[...]).astype(jnp.float32)
    b = jnp.zeros((_R, _W), jnp.int32)
    for i in range(_NBINS):
        b = b + (conf > bv_ref[i]).astype(jnp.int32)
    conf_ref[...] = conf
    acc_ref[...] = acc
    bin_ref[...] = b


_tc_call = pl.pallas_call(
    _tc_body,
    grid=(_H // _R,),
    in_specs=[
        pl.BlockSpec(memory_space=pltpu.SMEM),
        pl.BlockSpec((1, _C // 2, _R, _W), lambda i: (0, 0, i, 0)),
        pl.BlockSpec((1, _C // 2, _R, _W), lambda i: (1, 0, i, 0)),
        pl.BlockSpec((_R, _W), lambda i: (i, 0)),
    ],
    out_specs=[
        pl.BlockSpec((_R, _W), lambda i: (i, 0)),
        pl.BlockSpec((_R, _W), lambda i: (i, 0)),
        pl.BlockSpec((_R, _W), lambda i: (i, 0)),
    ],
    out_shape=[
        jax.ShapeDtypeStruct((_H, _W), jnp.float32),
        jax.ShapeDtypeStruct((_H, _W), jnp.float32),
        jax.ShapeDtypeStruct((_H, _W), jnp.int32),
    ],
)


def _sc_body(bin_hbm, conf_hbm, acc_hbm, out_hbm, bin_v, conf_v, acc_v,
             hist_v, red_v):
    wid = lax.axis_index("s") * 2 + lax.axis_index("c")
    base = wid * _PW
    pltpu.sync_copy(bin_hbm.at[pl.ds(base, _PW)], bin_v)
    pltpu.sync_copy(conf_hbm.at[pl.ds(base, _PW)], conf_v)
    pltpu.sync_copy(acc_hbm.at[pl.ds(base, _PW)], acc_v)
    zero = jnp.zeros((16,), jnp.float32)
    for j in range(48):
        hist_v[pl.ds(16 * j, 16)] = zero
    ones = jnp.ones((16,), jnp.float32)
    lane48 = jnp.arange(16, dtype=jnp.int32) * 48
    k16 = jnp.full((16,), 16, jnp.int32)
    k32 = jnp.full((16,), 32, jnp.int32)

    def body(i, carry):
        off = i * 16
        b = bin_v[pl.ds(off, 16)] + lane48
        cv = conf_v[pl.ds(off, 16)]
        av = acc_v[pl.ds(off, 16)]
        plsc.addupdate_scatter(hist_v, [b], ones)
        plsc.addupdate_scatter(hist_v, [b + k16], cv)
        plsc.addupdate_scatter(hist_v, [b + k32], av)
        return carry

    lax.fori_loop(0, _PW // 16, body, 0)
    for g in range(3):
        s = hist_v[pl.ds(16 * g, 16)]
        for l in range(1, 16):
            s = s + hist_v[pl.ds(48 * l + 16 * g, 16)]
        red_v[pl.ds(16 * g, 16)] = s
    pltpu.sync_copy(red_v, out_hbm.at[wid])


def _make_sc_hist():
    return pl.kernel(
        _sc_body,
        mesh=plsc.VectorSubcoreMesh(core_axis_name="c", subcore_axis_name="s"),
        compiler_params=pltpu.CompilerParams(needs_layout_passes=False),
        out_type=jax.ShapeDtypeStruct((_NW, 48), jnp.float32),
        scratch_types=[
            pltpu.VMEM((_PW,), jnp.int32),
            pltpu.VMEM((_PW,), jnp.float32),
            pltpu.VMEM((_PW,), jnp.float32),
            pltpu.VMEM((16 * 48,), jnp.float32),
            pltpu.VMEM((48,), jnp.float32),
        ],
    )


def kernel(probs, labels):
    p = probs.reshape(2, _C // 2, _H, _W)
    lab = labels.reshape(_H, _W)
    bvals = jnp.linspace(0.0, 1.0, _NBINS + 1)
    conf, acc, binid = _tc_call(bvals, p, p, lab)
    hist = _make_sc_hist()(binid.reshape(_PIX), conf.reshape(_PIX), acc.reshape(_PIX))
    h = jnp.sum(hist, axis=0)
    return h[1:16], h[17:32], h[33:48]


# speedup vs baseline: 1.0497x; 1.0497x over previous
import jax
import jax.numpy as jnp
from jax import lax
from jax.experimental import pallas as pl
from jax.experimental.pallas import tpu as pltpu
from jax.experimental.pallas import tpu_sc as plsc

_NBINS = 15
_C = 96
_H = 512
_W = 512
_R = 64
_PIX = _H * _W
_NW = 32
_PW = _PIX // _NW


def _tc_body(bv_ref, probs_ref, lab_ref, conf_ref, acc_ref, bin_ref):
    x = probs_ref[...]
    conf = jnp.max(x, axis=0)
    pred = jnp.argmax(x, axis=0).astype(jnp.int32)
    acc = (pred == lab_ref[...]).astype(jnp.float32)
    b = jnp.zeros((_R, _W), jnp.int32)
    for i in range(_NBINS):
        b = b + (conf > bv_ref[i]).astype(jnp.int32)
    conf_ref[...] = conf
    acc_ref[...] = acc
    bin_ref[...] = b


_tc_call = pl.pallas_call(
    _tc_body,
    grid=(_H // _R,),
    in_specs=[
        pl.BlockSpec(memory_space=pltpu.SMEM),
        pl.BlockSpec((_C, _R, _W), lambda i: (0, i, 0)),
        pl.BlockSpec((_R, _W), lambda i: (i, 0)),
    ],
    out_specs=[
        pl.BlockSpec((_R, _W), lambda i: (i, 0)),
        pl.BlockSpec((_R, _W), lambda i: (i, 0)),
        pl.BlockSpec((_R, _W), lambda i: (i, 0)),
    ],
    out_shape=[
        jax.ShapeDtypeStruct((_H, _W), jnp.float32),
        jax.ShapeDtypeStruct((_H, _W), jnp.float32),
        jax.ShapeDtypeStruct((_H, _W), jnp.int32),
    ],
)


_UNROLL = 4


def _sc_body(bin_hbm, conf_hbm, acc_hbm, out_hbm, bin_v, conf_v, acc_v,
             hist_v, red_v, sem):
    wid = lax.axis_index("s") * 2 + lax.axis_index("c")
    base = wid * _PW
    cp_b = pltpu.async_copy(bin_hbm.at[pl.ds(base, _PW)], bin_v, sem)
    cp_c = pltpu.async_copy(conf_hbm.at[pl.ds(base, _PW)], conf_v, sem)
    cp_a = pltpu.async_copy(acc_hbm.at[pl.ds(base, _PW)], acc_v, sem)
    zero = jnp.zeros((16,), jnp.float32)
    for j in range(48):
        hist_v[pl.ds(16 * j, 16)] = zero
    cp_b.wait()
    cp_c.wait()
    cp_a.wait()
    ones = jnp.ones((16,), jnp.float32)
    lane48 = jnp.arange(16, dtype=jnp.int32) * 48
    k16 = jnp.full((16,), 16, jnp.int32)
    k32 = jnp.full((16,), 32, jnp.int32)

    def body(i, carry):
        for u in range(_UNROLL):
            off = i * (16 * _UNROLL) + u * 16
            b = bin_v[pl.ds(off, 16)] + lane48
            cv = conf_v[pl.ds(off, 16)]
            av = acc_v[pl.ds(off, 16)]
            plsc.addupdate_scatter(hist_v, [b], ones)
            plsc.addupdate_scatter(hist_v, [b + k16], cv)
            plsc.addupdate_scatter(hist_v, [b + k32], av)
        return carry

    lax.fori_loop(0, _PW // (16 * _UNROLL), body, 0)
    for g in range(3):
        s = hist_v[pl.ds(16 * g, 16)]
        for l in range(1, 16):
            s = s + hist_v[pl.ds(48 * l + 16 * g, 16)]
        red_v[pl.ds(16 * g, 16)] = s
    pltpu.sync_copy(red_v, out_hbm.at[wid])


def _make_sc_hist():
    return pl.kernel(
        _sc_body,
        mesh=plsc.VectorSubcoreMesh(core_axis_name="c", subcore_axis_name="s"),
        compiler_params=pltpu.CompilerParams(needs_layout_passes=False),
        out_type=jax.ShapeDtypeStruct((_NW, 48), jnp.float32),
        scratch_types=[
            pltpu.VMEM((_PW,), jnp.int32),
            pltpu.VMEM((_PW,), jnp.float32),
            pltpu.VMEM((_PW,), jnp.float32),
            pltpu.VMEM((16 * 48,), jnp.float32),
            pltpu.VMEM((48,), jnp.float32),
            pltpu.SemaphoreType.DMA,
        ],
    )


def kernel(probs, labels):
    p = probs.reshape(_C, _H, _W)
    lab = labels.reshape(_H, _W)
    bvals = jnp.linspace(0.0, 1.0, _NBINS + 1)
    conf, acc, binid = _tc_call(bvals, p, lab)
    hist = _make_sc_hist()(binid.reshape(_PIX), conf.reshape(_PIX), acc.reshape(_PIX))
    h = jnp.sum(hist, axis=0)
    return h[1:16], h[17:32], h[33:48]


# speedup vs baseline: 1.1527x; 1.0981x over previous
import jax
import jax.numpy as jnp
from jax import lax
from jax.experimental import pallas as pl
from jax.experimental.pallas import tpu as pltpu
from jax.experimental.pallas import tpu_sc as plsc

_NBINS = 15
_C = 96
_H = 512
_W = 512
_R = 64
_PIX = _H * _W
_NW = 32
_PW = _PIX // _NW


def _tc_body(bv_ref, probs_ref, lab_ref, conf_ref, acc_ref, bin_ref):
    x = probs_ref[...]
    conf = jnp.max(x, axis=0)
    pred = jnp.argmax(x, axis=0).astype(jnp.int32)
    acc = (pred == lab_ref[...]).astype(jnp.float32)
    b = jnp.zeros((_R, _W), jnp.int32)
    for i in range(_NBINS):
        b = b + (conf > bv_ref[i]).astype(jnp.int32)
    conf_ref[...] = conf
    acc_ref[...] = acc
    bin_ref[...] = b


_tc_call = pl.pallas_call(
    _tc_body,
    grid=(_H // _R,),
    in_specs=[
        pl.BlockSpec(memory_space=pltpu.SMEM),
        pl.BlockSpec((_C, _R, _W), lambda i: (0, i, 0)),
        pl.BlockSpec((_R, _W), lambda i: (i, 0)),
    ],
    out_specs=[
        pl.BlockSpec((_R, _W), lambda i: (i, 0)),
        pl.BlockSpec((_R, _W), lambda i: (i, 0)),
        pl.BlockSpec((_R, _W), lambda i: (i, 0)),
    ],
    out_shape=[
        jax.ShapeDtypeStruct((_H, _W), jnp.float32),
        jax.ShapeDtypeStruct((_H, _W), jnp.float32),
        jax.ShapeDtypeStruct((_H, _W), jnp.int32),
    ],
)


_UNROLL = 4


def _sc_body(bin_hbm, conf_hbm, acc_hbm, out_hbm, bin_v, conf_v, acc_v,
             hist_v, red_v, sem):
    wid = lax.axis_index("s") * 2 + lax.axis_index("c")
    base = wid * _PW
    cp_b = pltpu.async_copy(bin_hbm.at[pl.ds(base, _PW)], bin_v, sem)
    cp_c = pltpu.async_copy(conf_hbm.at[pl.ds(base, _PW)], conf_v, sem)
    cp_a = pltpu.async_copy(acc_hbm.at[pl.ds(base, _PW)], acc_v, sem)
    zero = jnp.zeros((16,), jnp.float32)
    for j in range(48):
        hist_v[pl.ds(16 * j, 16)] = zero
    cp_b.wait()
    cp_c.wait()
    cp_a.wait()
    ones = jnp.ones((16,), jnp.float32)
    lane49 = jnp.arange(16, dtype=jnp.int32) * 49
    k16 = jnp.full((16,), 16, jnp.int32)
    k32 = jnp.full((16,), 32, jnp.int32)

    def body(i, carry):
        for u in range(_UNROLL):
            off = i * (16 * _UNROLL) + u * 16
            b = bin_v[pl.ds(off, 16)] + lane49
            cv = conf_v[pl.ds(off, 16)]
            av = acc_v[pl.ds(off, 16)]
            plsc.addupdate_scatter(hist_v, [b], ones)
            plsc.addupdate_scatter(hist_v, [b + k16], cv)
            plsc.addupdate_scatter(hist_v, [b + k32], av)
        return carry

    lax.fori_loop(0, _PW // (16 * _UNROLL), body, 0)
    iota16 = jnp.arange(16, dtype=jnp.int32)
    for g in range(3):
        s = plsc.load_gather(hist_v, [iota16 + 16 * g])
        for l in range(1, 16):
            s = s + plsc.load_gather(hist_v, [iota16 + (49 * l + 16 * g)])
        red_v[pl.ds(16 * g, 16)] = s
    pltpu.sync_copy(red_v, out_hbm.at[wid])


def _make_sc_hist():
    return pl.kernel(
        _sc_body,
        mesh=plsc.VectorSubcoreMesh(core_axis_name="c", subcore_axis_name="s"),
        compiler_params=pltpu.CompilerParams(needs_layout_passes=False),
        out_type=jax.ShapeDtypeStruct((_NW, 48), jnp.float32),
        scratch_types=[
            pltpu.VMEM((_PW,), jnp.int32),
            pltpu.VMEM((_PW,), jnp.float32),
            pltpu.VMEM((_PW,), jnp.float32),
            pltpu.VMEM((16 * 49,), jnp.float32),
            pltpu.VMEM((48,), jnp.float32),
            pltpu.SemaphoreType.DMA,
        ],
    )


def kernel(probs, labels):
    p = probs.reshape(_C, _H, _W)
    lab = labels.reshape(_H, _W)
    bvals = jnp.linspace(0.0, 1.0, _NBINS + 1)
    conf, acc, binid = _tc_call(bvals, p, lab)
    hist = _make_sc_hist()(binid.reshape(_PIX), conf.reshape(_PIX), acc.reshape(_PIX))
    h = jnp.sum(hist, axis=0)
    return h[1:16], h[17:32], h[33:48]
